# baseline (speedup 1.0000x reference)
import functools

import jax
import jax.numpy as jnp
from jax.experimental import pallas as pl
from jax.experimental.pallas import tpu as pltpu


def _upsample_kernel(x_ref, w_ref, b_ref, o_ref):
    cin = x_ref.shape[1]
    h = o_ref.shape[1]
    w_dim = o_ref.shape[3]
    two_cout = o_ref.shape[4]
    xc = x_ref[0].astype(jnp.bfloat16).reshape(cin, h * w_dim)
    y = jax.lax.dot_general(
        xc, w_ref[...], (((0,), (0,)), ((), ())),
        preferred_element_type=jnp.float32,
    ) + b_ref[...]
    yb = y.astype(jnp.bfloat16)
    for a in range(2):
        ya = yb[:, a * two_cout:(a + 1) * two_cout]
        o_ref[0, :, a, :, :] = ya.reshape(h, w_dim, two_cout)


def _double_conv_kernel(x_ref, w1_ref, s1_ref, t1_ref, w2_ref, s2_ref, t2_ref,
                        o_ref, xs_ref, *, wo):
    _, hw, c = x_ref.shape
    cout = o_ref.shape[1]

    x_cm = x_ref[0].T

    col = jax.lax.broadcasted_iota(jnp.int32, (1, hw), 1) % wo
    left_edge = col == 0
    right_edge = col == wo - 1

    def shift(xs, s):
        if s > 0:
            return jnp.concatenate(
                [xs[:, s:], jnp.zeros((xs.shape[0], s), xs.dtype)], axis=1)
        if s < 0:
            return jnp.concatenate(
                [jnp.zeros((xs.shape[0], -s), xs.dtype), xs[:, :s]], axis=1)
        return xs

    def conv(xcm, w_ref_, s_, t_):
        zero = jnp.zeros_like(xcm)
        pick = {-1: jnp.where(right_edge, zero, xcm),
                0: xcm,
                1: jnp.where(left_edge, zero, xcm)}
        acc = jnp.zeros((cout, hw), jnp.float32)
        k = 0
        for dh in (-1, 0, 1):
            for dw in (-1, 0, 1):
                xs_ref[k % 2] = shift(pick[dw], dh * wo + dw)
                acc = acc + jnp.dot(w_ref_[k], xs_ref[k % 2],
                                    preferred_element_type=jnp.float32)
                k += 1
        return jnp.maximum(acc * s_ + t_, 0.0)

    y1 = conv(x_cm, w1_ref, s1_ref[...], t1_ref[...])
    y2 = conv(y1.astype(jnp.bfloat16), w2_ref, s2_ref[...], t2_ref[...])
    o_ref[0] = y2.reshape(cout, hw // wo, wo)


def kernel(x_nchw, w_up, b_up, conv1_w, conv1_scale, conv1_shift,
           conv2_w, conv2_scale, conv2_shift):
    n, cin, h, w = x_nchw.shape
    cout = w_up.shape[1]
    ho, wo = 2 * h, 2 * w

    w2d = jnp.transpose(w_up, (0, 2, 3, 1)).reshape(cin, 4 * cout)
    w2d = w2d.astype(jnp.bfloat16)
    b2d = jnp.tile(b_up, 4).reshape(1, 4 * cout)
    w1 = jnp.transpose(conv1_w, (0, 1, 3, 2)).reshape(9, cout, cout)
    w1 = w1.astype(jnp.bfloat16)
    w2 = jnp.transpose(conv2_w, (0, 1, 3, 2)).reshape(9, cout, cout)
    w2 = w2.astype(jnp.bfloat16)
    s1 = conv1_scale.reshape(cout, 1)
    t1 = conv1_shift.reshape(cout, 1)
    s2 = conv2_scale.reshape(cout, 1)
    t2 = conv2_shift.reshape(cout, 1)

    up = pl.pallas_call(
        _upsample_kernel,
        out_shape=jax.ShapeDtypeStruct((n, h, 2, w, 2 * cout), jnp.bfloat16),
        grid=(n,),
        in_specs=[
            pl.BlockSpec((1, cin, h, w), lambda i: (i, 0, 0, 0)),
            pl.BlockSpec((cin, 4 * cout), lambda i: (0, 0)),
            pl.BlockSpec((1, 4 * cout), lambda i: (0, 0)),
        ],
        out_specs=pl.BlockSpec((1, h, 2, w, 2 * cout),
                               lambda i: (i, 0, 0, 0, 0)),
        compiler_params=pltpu.CompilerParams(
            dimension_semantics=("parallel",)),
    )(x_nchw, w2d, b2d)
    fine = up.reshape(n, ho * wo, cout)

    out = pl.pallas_call(
        functools.partial(_double_conv_kernel, wo=wo),
        out_shape=jax.ShapeDtypeStruct((n, cout, ho, wo), jnp.float32),
        grid=(n,),
        in_specs=[
            pl.BlockSpec((1, ho * wo, cout), lambda i: (i, 0, 0)),
            pl.BlockSpec((9, cout, cout), lambda i: (0, 0, 0)),
            pl.BlockSpec((cout, 1), lambda i: (0, 0)),
            pl.BlockSpec((cout, 1), lambda i: (0, 0)),
            pl.BlockSpec((9, cout, cout), lambda i: (0, 0, 0)),
            pl.BlockSpec((cout, 1), lambda i: (0, 0)),
            pl.BlockSpec((cout, 1), lambda i: (0, 0)),
        ],
        out_specs=pl.BlockSpec((1, cout, ho, wo), lambda i: (i, 0, 0, 0)),
        scratch_shapes=[pltpu.VMEM((2, cout, ho * wo), jnp.bfloat16)],
        compiler_params=pltpu.CompilerParams(
            dimension_semantics=("parallel",)),
    )(fine, w1, s1, t1, w2, s2, t2)

    return out


# speedup vs baseline: 1.1731x; 1.1731x over previous
import functools

import jax
import jax.numpy as jnp
from jax.experimental import pallas as pl
from jax.experimental.pallas import tpu as pltpu


def _upsample_kernel(x_ref, w_ref, b_ref, o_ref):
    cin = x_ref.shape[1]
    h = o_ref.shape[1]
    w_dim = o_ref.shape[3]
    two_cout = o_ref.shape[4]
    xc = x_ref[0].astype(jnp.bfloat16).reshape(cin, h * w_dim)
    y = jax.lax.dot_general(
        xc, w_ref[...], (((0,), (0,)), ((), ())),
        preferred_element_type=jnp.float32,
    ) + b_ref[...]
    yb = y.astype(jnp.bfloat16)
    for a in range(2):
        ya = yb[:, a * two_cout:(a + 1) * two_cout]
        o_ref[0, :, a, :, :] = ya.reshape(h, w_dim, two_cout)


def _double_conv_kernel(x_ref, w1_ref, s1_ref, t1_ref, w2_ref, s2_ref, t2_ref,
                        o_ref, xs_ref, *, wo):
    _, hw, c = x_ref.shape
    cout = o_ref.shape[1]

    x_cm = x_ref[0].T

    col = jax.lax.broadcasted_iota(jnp.int32, (1, hw), 1) % wo
    left_edge = col == 0
    right_edge = col == wo - 1

    def shift(xs, s):
        if s > 0:
            return jnp.concatenate(
                [xs[:, s:], jnp.zeros((xs.shape[0], s), xs.dtype)], axis=1)
        if s < 0:
            return jnp.concatenate(
                [jnp.zeros((xs.shape[0], -s), xs.dtype), xs[:, :s]], axis=1)
        return xs

    def conv(xcm, w_ref_, s_, t_):
        zero = jnp.zeros_like(xcm)
        pick = {-1: jnp.where(right_edge, zero, xcm),
                0: xcm,
                1: jnp.where(left_edge, zero, xcm)}
        acc = jnp.zeros((cout, hw), jnp.float32)
        k = 0
        for dh in (-1, 0, 1):
            for dw in (-1, 0, 1):
                xs_ref[k % 2] = shift(pick[dw], dh * wo + dw)
                acc = acc + jnp.dot(w_ref_[k], xs_ref[k % 2],
                                    preferred_element_type=jnp.float32)
                k += 1
        return jnp.maximum(acc * s_ + t_, 0.0)

    y1 = conv(x_cm, w1_ref, s1_ref[...], t1_ref[...])
    y2 = conv(y1.astype(jnp.bfloat16), w2_ref, s2_ref[...], t2_ref[...])
    o_ref[0] = y2


def kernel(x_nchw, w_up, b_up, conv1_w, conv1_scale, conv1_shift,
           conv2_w, conv2_scale, conv2_shift):
    n, cin, h, w = x_nchw.shape
    cout = w_up.shape[1]
    ho, wo = 2 * h, 2 * w

    w2d = jnp.transpose(w_up, (0, 2, 3, 1)).reshape(cin, 4 * cout)
    w2d = w2d.astype(jnp.bfloat16)
    b2d = jnp.tile(b_up, 4).reshape(1, 4 * cout)
    w1 = jnp.transpose(conv1_w, (0, 1, 3, 2)).reshape(9, cout, cout)
    w1 = w1.astype(jnp.bfloat16)
    w2 = jnp.transpose(conv2_w, (0, 1, 3, 2)).reshape(9, cout, cout)
    w2 = w2.astype(jnp.bfloat16)
    s1 = conv1_scale.reshape(cout, 1)
    t1 = conv1_shift.reshape(cout, 1)
    s2 = conv2_scale.reshape(cout, 1)
    t2 = conv2_shift.reshape(cout, 1)

    up = pl.pallas_call(
        _upsample_kernel,
        out_shape=jax.ShapeDtypeStruct((n, h, 2, w, 2 * cout), jnp.bfloat16),
        grid=(n,),
        in_specs=[
            pl.BlockSpec((1, cin, h, w), lambda i: (i, 0, 0, 0)),
            pl.BlockSpec((cin, 4 * cout), lambda i: (0, 0)),
            pl.BlockSpec((1, 4 * cout), lambda i: (0, 0)),
        ],
        out_specs=pl.BlockSpec((1, h, 2, w, 2 * cout),
                               lambda i: (i, 0, 0, 0, 0)),
        compiler_params=pltpu.CompilerParams(
            dimension_semantics=("parallel",)),
    )(x_nchw, w2d, b2d)
    fine = up.reshape(n, ho * wo, cout)

    out = pl.pallas_call(
        functools.partial(_double_conv_kernel, wo=wo),
        out_shape=jax.ShapeDtypeStruct((n, cout, ho * wo), jnp.float32),
        grid=(n,),
        in_specs=[
            pl.BlockSpec((1, ho * wo, cout), lambda i: (i, 0, 0)),
            pl.BlockSpec((9, cout, cout), lambda i: (0, 0, 0)),
            pl.BlockSpec((cout, 1), lambda i: (0, 0)),
            pl.BlockSpec((cout, 1), lambda i: (0, 0)),
            pl.BlockSpec((9, cout, cout), lambda i: (0, 0, 0)),
            pl.BlockSpec((cout, 1), lambda i: (0, 0)),
            pl.BlockSpec((cout, 1), lambda i: (0, 0)),
        ],
        out_specs=pl.BlockSpec((1, cout, ho * wo), lambda i: (i, 0, 0)),
        scratch_shapes=[pltpu.VMEM((2, cout, ho * wo), jnp.bfloat16)],
        compiler_params=pltpu.CompilerParams(
            dimension_semantics=("parallel",)),
    )(fine, w1, s1, t1, w2, s2, t2)

    return out.reshape(n, cout, ho, wo)


# speedup vs baseline: 1.5977x; 1.3619x over previous
import functools

import jax
import jax.numpy as jnp
from jax.experimental import pallas as pl
from jax.experimental.pallas import tpu as pltpu


def _upsample_kernel(x_ref, w_ref, b_ref, o_ref):
    cin = x_ref.shape[1]
    h = o_ref.shape[1]
    w_dim = o_ref.shape[3]
    two_cout = o_ref.shape[4]
    xc = x_ref[0].astype(jnp.bfloat16)
    y = jax.lax.dot_general(
        xc, w_ref[...], (((0,), (0,)), ((), ())),
        preferred_element_type=jnp.float32,
    ) + b_ref[...]
    yb = y.astype(jnp.bfloat16)
    for a in range(2):
        ya = yb[:, a * two_cout:(a + 1) * two_cout]
        o_ref[0, :, a, :, :] = ya.reshape(h, w_dim, two_cout)


def _double_conv_kernel(x_ref, w1_ref, s1_ref, t1_ref, w2_ref, s2_ref, t2_ref,
                        o_ref, xs_ref, *, wo):
    _, hw, c = x_ref.shape
    cout = o_ref.shape[1]

    x_cm = x_ref[0].T

    col = jax.lax.broadcasted_iota(jnp.int32, (1, hw), 1) % wo
    left_edge = col == 0
    right_edge = col == wo - 1

    def shift(xs, s):
        if s > 0:
            return jnp.concatenate(
                [xs[:, s:], jnp.zeros((xs.shape[0], s), xs.dtype)], axis=1)
        if s < 0:
            return jnp.concatenate(
                [jnp.zeros((xs.shape[0], -s), xs.dtype), xs[:, :s]], axis=1)
        return xs

    def conv(xcm, w_ref_, s_, t_):
        zero = jnp.zeros_like(xcm)
        pick = {-1: jnp.where(right_edge, zero, xcm),
                0: xcm,
                1: jnp.where(left_edge, zero, xcm)}
        acc = jnp.zeros((cout, hw), jnp.float32)
        for dh_i, dh in enumerate((-1, 0, 1)):
            buf = xs_ref.at[dh_i % 2]
            for dw_i, dw in enumerate((-1, 0, 1)):
                buf[dw_i * c:(dw_i + 1) * c] = shift(pick[dw], dh * wo + dw)
            acc = acc + jnp.dot(w_ref_[dh_i], buf[...],
                                preferred_element_type=jnp.float32)
        return jnp.maximum(acc * s_ + t_, 0.0)

    y1 = conv(x_cm, w1_ref, s1_ref[...], t1_ref[...])
    y2 = conv(y1.astype(jnp.bfloat16), w2_ref, s2_ref[...], t2_ref[...])
    o_ref[0] = y2


def kernel(x_nchw, w_up, b_up, conv1_w, conv1_scale, conv1_shift,
           conv2_w, conv2_scale, conv2_shift):
    n, cin, h, w = x_nchw.shape
    cout = w_up.shape[1]
    ho, wo = 2 * h, 2 * w

    w2d = jnp.transpose(w_up, (0, 2, 3, 1)).reshape(cin, 4 * cout)
    w2d = w2d.astype(jnp.bfloat16)
    b2d = jnp.tile(b_up, 4).reshape(1, 4 * cout)
    w1 = jnp.transpose(conv1_w, (0, 3, 1, 2)).reshape(3, cout, 3 * cout)
    w1 = w1.astype(jnp.bfloat16)
    w2 = jnp.transpose(conv2_w, (0, 3, 1, 2)).reshape(3, cout, 3 * cout)
    w2 = w2.astype(jnp.bfloat16)
    s1 = conv1_scale.reshape(cout, 1)
    t1 = conv1_shift.reshape(cout, 1)
    s2 = conv2_scale.reshape(cout, 1)
    t2 = conv2_shift.reshape(cout, 1)

    x_flat = x_nchw.reshape(n, cin, h * w)

    up = pl.pallas_call(
        _upsample_kernel,
        out_shape=jax.ShapeDtypeStruct((n, h, 2, w, 2 * cout), jnp.bfloat16),
        grid=(n,),
        in_specs=[
            pl.BlockSpec((1, cin, h * w), lambda i: (i, 0, 0)),
            pl.BlockSpec((cin, 4 * cout), lambda i: (0, 0)),
            pl.BlockSpec((1, 4 * cout), lambda i: (0, 0)),
        ],
        out_specs=pl.BlockSpec((1, h, 2, w, 2 * cout),
                               lambda i: (i, 0, 0, 0, 0)),
        compiler_params=pltpu.CompilerParams(
            dimension_semantics=("parallel",)),
    )(x_flat, w2d, b2d)
    fine = up.reshape(n, ho * wo, cout)

    out = pl.pallas_call(
        functools.partial(_double_conv_kernel, wo=wo),
        out_shape=jax.ShapeDtypeStruct((n, cout, ho * wo), jnp.float32),
        grid=(n,),
        in_specs=[
            pl.BlockSpec((1, ho * wo, cout), lambda i: (i, 0, 0)),
            pl.BlockSpec((3, cout, 3 * cout), lambda i: (0, 0, 0)),
            pl.BlockSpec((cout, 1), lambda i: (0, 0)),
            pl.BlockSpec((cout, 1), lambda i: (0, 0)),
            pl.BlockSpec((3, cout, 3 * cout), lambda i: (0, 0, 0)),
            pl.BlockSpec((cout, 1), lambda i: (0, 0)),
            pl.BlockSpec((cout, 1), lambda i: (0, 0)),
        ],
        out_specs=pl.BlockSpec((1, cout, ho * wo), lambda i: (i, 0, 0)),
        scratch_shapes=[pltpu.VMEM((2, 3 * cout, ho * wo), jnp.bfloat16)],
        compiler_params=pltpu.CompilerParams(
            dimension_semantics=("parallel",)),
    )(fine, w1, s1, t1, w2, s2, t2)

    return out.reshape(n, cout, ho, wo)


# speedup vs baseline: 1.6429x; 1.0283x over previous
import functools

import jax
import jax.numpy as jnp
from jax.experimental import pallas as pl
from jax.experimental.pallas import tpu as pltpu


def _upsample_kernel(x_ref, w_ref, b_ref, o_ref):
    bsz = x_ref.shape[0]
    h = o_ref.shape[1]
    w_dim = o_ref.shape[3]
    two_cout = o_ref.shape[4]
    for img in range(bsz):
        xc = x_ref[img].astype(jnp.bfloat16)
        y = jax.lax.dot_general(
            xc, w_ref[...], (((0,), (0,)), ((), ())),
            preferred_element_type=jnp.float32,
        ) + b_ref[...]
        yb = y.astype(jnp.bfloat16)
        for a in range(2):
            ya = yb[:, a * two_cout:(a + 1) * two_cout]
            o_ref[img, :, a, :, :] = ya.reshape(h, w_dim, two_cout)


def _double_conv_kernel(x_ref, w1_ref, s1_ref, t1_ref, w2_ref, s2_ref, t2_ref,
                        o_ref, xs_ref, *, wo):
    _, hw, c = x_ref.shape
    cout = o_ref.shape[1]

    x_cm = x_ref[0].T

    col = jax.lax.broadcasted_iota(jnp.int32, (1, hw), 1) % wo
    left_edge = col == 0
    right_edge = col == wo - 1

    def shift(xs, s):
        if s > 0:
            return jnp.concatenate(
                [xs[:, s:], jnp.zeros((xs.shape[0], s), xs.dtype)], axis=1)
        if s < 0:
            return jnp.concatenate(
                [jnp.zeros((xs.shape[0], -s), xs.dtype), xs[:, :s]], axis=1)
        return xs

    def conv(xcm, w_ref_, s_, t_):
        zero = jnp.zeros_like(xcm)
        pick = {-1: jnp.where(right_edge, zero, xcm),
                0: xcm,
                1: jnp.where(left_edge, zero, xcm)}
        acc = jnp.zeros((cout, hw), jnp.float32)
        for dh_i, dh in enumerate((-1, 0, 1)):
            buf = xs_ref.at[dh_i % 2]
            for dw_i, dw in enumerate((-1, 0, 1)):
                buf[dw_i * c:(dw_i + 1) * c] = shift(pick[dw], dh * wo + dw)
            acc = acc + jnp.dot(w_ref_[dh_i], buf[...],
                                preferred_element_type=jnp.float32)
        return jnp.maximum(acc * s_ + t_, 0.0)

    y1 = conv(x_cm, w1_ref, s1_ref[...], t1_ref[...])
    y2 = conv(y1.astype(jnp.bfloat16), w2_ref, s2_ref[...], t2_ref[...])
    o_ref[0] = y2


def kernel(x_nchw, w_up, b_up, conv1_w, conv1_scale, conv1_shift,
           conv2_w, conv2_scale, conv2_shift):
    n, cin, h, w = x_nchw.shape
    cout = w_up.shape[1]
    ho, wo = 2 * h, 2 * w

    w2d = jnp.transpose(w_up, (0, 2, 3, 1)).reshape(cin, 4 * cout)
    w2d = w2d.astype(jnp.bfloat16)
    b2d = jnp.tile(b_up, 4).reshape(1, 4 * cout)
    w1 = jnp.transpose(conv1_w, (0, 3, 1, 2)).reshape(3, cout, 3 * cout)
    w1 = w1.astype(jnp.bfloat16)
    w2 = jnp.transpose(conv2_w, (0, 3, 1, 2)).reshape(3, cout, 3 * cout)
    w2 = w2.astype(jnp.bfloat16)
    s1 = conv1_scale.reshape(cout, 1)
    t1 = conv1_shift.reshape(cout, 1)
    s2 = conv2_scale.reshape(cout, 1)
    t2 = conv2_shift.reshape(cout, 1)

    x_flat = x_nchw.reshape(n, cin, h * w)

    bsz = 2 if n % 2 == 0 else 1
    up = pl.pallas_call(
        _upsample_kernel,
        out_shape=jax.ShapeDtypeStruct((n, h, 2, w, 2 * cout), jnp.bfloat16),
        grid=(n // bsz,),
        in_specs=[
            pl.BlockSpec((bsz, cin, h * w), lambda i: (i, 0, 0)),
            pl.BlockSpec((cin, 4 * cout), lambda i: (0, 0)),
            pl.BlockSpec((1, 4 * cout), lambda i: (0, 0)),
        ],
        out_specs=pl.BlockSpec((bsz, h, 2, w, 2 * cout),
                               lambda i: (i, 0, 0, 0, 0)),
        compiler_params=pltpu.CompilerParams(
            dimension_semantics=("parallel",)),
    )(x_flat, w2d, b2d)
    fine = up.reshape(n, ho * wo, cout)

    out = pl.pallas_call(
        functools.partial(_double_conv_kernel, wo=wo),
        out_shape=jax.ShapeDtypeStruct((n, cout, ho * wo), jnp.float32),
        grid=(n,),
        in_specs=[
            pl.BlockSpec((1, ho * wo, cout), lambda i: (i, 0, 0)),
            pl.BlockSpec((3, cout, 3 * cout), lambda i: (0, 0, 0)),
            pl.BlockSpec((cout, 1), lambda i: (0, 0)),
            pl.BlockSpec((cout, 1), lambda i: (0, 0)),
            pl.BlockSpec((3, cout, 3 * cout), lambda i: (0, 0, 0)),
            pl.BlockSpec((cout, 1), lambda i: (0, 0)),
            pl.BlockSpec((cout, 1), lambda i: (0, 0)),
        ],
        out_specs=pl.BlockSpec((1, cout, ho * wo), lambda i: (i, 0, 0)),
        scratch_shapes=[pltpu.VMEM((2, 3 * cout, ho * wo), jnp.bfloat16)],
        compiler_params=pltpu.CompilerParams(
            dimension_semantics=("parallel",)),
    )(fine, w1, s1, t1, w2, s2, t2)

    return out.reshape(n, cout, ho, wo)


# speedup vs baseline: 1.9299x; 1.1747x over previous
import functools

import jax
import jax.numpy as jnp
from jax.experimental import pallas as pl
from jax.experimental.pallas import tpu as pltpu


def _upsample_kernel(x_ref, w_ref, b_ref, o_ref):
    bsz = x_ref.shape[0]
    h = o_ref.shape[1]
    w_dim = o_ref.shape[3]
    two_cout = o_ref.shape[4]
    for img in range(bsz):
        xc = x_ref[img].astype(jnp.bfloat16)
        y = jax.lax.dot_general(
            xc, w_ref[...], (((0,), (0,)), ((), ())),
            preferred_element_type=jnp.float32,
        ) + b_ref[...]
        yb = y.astype(jnp.bfloat16)
        for a in range(2):
            ya = yb[:, a * two_cout:(a + 1) * two_cout]
            o_ref[img, :, a, :, :] = ya.reshape(h, w_dim, two_cout)


def _double_conv_kernel(x_ref, w1_ref, s1_ref, t1_ref, w2_ref, s2_ref, t2_ref,
                        o_ref, xs_ref, *, wo):
    cout = o_ref.shape[1]
    hw = o_ref.shape[2]
    c = cout

    x_cm = x_ref[0].reshape(hw, c).T

    col = jax.lax.broadcasted_iota(jnp.int32, (1, hw), 1) % wo
    left_edge = col == 0
    right_edge = col == wo - 1

    def shift(xs, s):
        if s > 0:
            return jnp.concatenate(
                [xs[:, s:], jnp.zeros((xs.shape[0], s), xs.dtype)], axis=1)
        if s < 0:
            return jnp.concatenate(
                [jnp.zeros((xs.shape[0], -s), xs.dtype), xs[:, :s]], axis=1)
        return xs

    def conv(xcm, w_ref_, s_, t_):
        zero = jnp.zeros_like(xcm)
        pick = {-1: jnp.where(right_edge, zero, xcm),
                0: xcm,
                1: jnp.where(left_edge, zero, xcm)}
        acc = jnp.zeros((cout, hw), jnp.float32)
        for dh_i, dh in enumerate((-1, 0, 1)):
            buf = xs_ref.at[dh_i % 2]
            for dw_i, dw in enumerate((-1, 0, 1)):
                buf[dw_i * c:(dw_i + 1) * c] = shift(pick[dw], dh * wo + dw)
            acc = acc + jnp.dot(w_ref_[dh_i], buf[...],
                                preferred_element_type=jnp.float32)
        return jnp.maximum(acc * s_ + t_, 0.0)

    y1 = conv(x_cm, w1_ref, s1_ref[...], t1_ref[...])
    y2 = conv(y1.astype(jnp.bfloat16), w2_ref, s2_ref[...], t2_ref[...])
    o_ref[0] = y2


def kernel(x_nchw, w_up, b_up, conv1_w, conv1_scale, conv1_shift,
           conv2_w, conv2_scale, conv2_shift):
    n, cin, h, w = x_nchw.shape
    cout = w_up.shape[1]
    ho, wo = 2 * h, 2 * w

    w2d = jnp.transpose(w_up, (0, 2, 3, 1)).reshape(cin, 4 * cout)
    w2d = w2d.astype(jnp.bfloat16)
    b2d = jnp.tile(b_up, 4).reshape(1, 4 * cout)
    w1 = jnp.transpose(conv1_w, (0, 3, 1, 2)).reshape(3, cout, 3 * cout)
    w1 = w1.astype(jnp.bfloat16)
    w2 = jnp.transpose(conv2_w, (0, 3, 1, 2)).reshape(3, cout, 3 * cout)
    w2 = w2.astype(jnp.bfloat16)
    s1 = conv1_scale.reshape(cout, 1)
    t1 = conv1_shift.reshape(cout, 1)
    s2 = conv2_scale.reshape(cout, 1)
    t2 = conv2_shift.reshape(cout, 1)

    x_flat = x_nchw.reshape(n, cin, h * w)

    bsz = 2 if n % 2 == 0 else 1
    up = pl.pallas_call(
        _upsample_kernel,
        out_shape=jax.ShapeDtypeStruct((n, h, 2, w, 2 * cout), jnp.bfloat16),
        grid=(n // bsz,),
        in_specs=[
            pl.BlockSpec((bsz, cin, h * w), lambda i: (i, 0, 0)),
            pl.BlockSpec((cin, 4 * cout), lambda i: (0, 0)),
            pl.BlockSpec((1, 4 * cout), lambda i: (0, 0)),
        ],
        out_specs=pl.BlockSpec((bsz, h, 2, w, 2 * cout),
                               lambda i: (i, 0, 0, 0, 0)),
        compiler_params=pltpu.CompilerParams(
            dimension_semantics=("parallel",)),
    )(x_flat, w2d, b2d)

    out = pl.pallas_call(
        functools.partial(_double_conv_kernel, wo=wo),
        out_shape=jax.ShapeDtypeStruct((n, cout, ho * wo), jnp.float32),
        grid=(n,),
        in_specs=[
            pl.BlockSpec((1, h, 2, w, 2 * cout), lambda i: (i, 0, 0, 0, 0)),
            pl.BlockSpec((3, cout, 3 * cout), lambda i: (0, 0, 0)),
            pl.BlockSpec((cout, 1), lambda i: (0, 0)),
            pl.BlockSpec((cout, 1), lambda i: (0, 0)),
            pl.BlockSpec((3, cout, 3 * cout), lambda i: (0, 0, 0)),
            pl.BlockSpec((cout, 1), lambda i: (0, 0)),
            pl.BlockSpec((cout, 1), lambda i: (0, 0)),
        ],
        out_specs=pl.BlockSpec((1, cout, ho * wo), lambda i: (i, 0, 0)),
        scratch_shapes=[pltpu.VMEM((2, 3 * cout, ho * wo), jnp.bfloat16)],
        compiler_params=pltpu.CompilerParams(
            dimension_semantics=("parallel",)),
    )(up, w1, s1, t1, w2, s2, t2)

    return out.reshape(n, cout, ho, wo)


# speedup vs baseline: 2.0184x; 1.0458x over previous
import functools

import jax
import jax.numpy as jnp
from jax.experimental import pallas as pl
from jax.experimental.pallas import tpu as pltpu


def _upsample_kernel(x_ref, w_ref, b_ref, o_ref):
    bsz = x_ref.shape[0]
    h = o_ref.shape[1]
    w_dim = o_ref.shape[3]
    two_cout = o_ref.shape[4]
    for img in range(bsz):
        xc = x_ref[img].astype(jnp.bfloat16)
        y = jax.lax.dot_general(
            xc, w_ref[...], (((0,), (0,)), ((), ())),
            preferred_element_type=jnp.float32,
        ) + b_ref[...]
        yb = y.astype(jnp.bfloat16)
        for a in range(2):
            ya = yb[:, a * two_cout:(a + 1) * two_cout]
            o_ref[img, :, a, :, :] = ya.reshape(h, w_dim, two_cout)


def _double_conv_kernel(x_ref, w1_ref, s1_ref, t1_ref, w2_ref, s2_ref, t2_ref,
                        o_ref, xs_ref, *, wo):
    bsz = o_ref.shape[0]
    cout = o_ref.shape[1]
    hw = o_ref.shape[2]
    c = cout

    col = jax.lax.broadcasted_iota(jnp.int32, (1, hw), 1) % wo
    left_edge = col == 0
    right_edge = col == wo - 1

    def shift(xs, s):
        if s > 0:
            return jnp.concatenate(
                [xs[:, s:], jnp.zeros((xs.shape[0], s), xs.dtype)], axis=1)
        if s < 0:
            return jnp.concatenate(
                [jnp.zeros((xs.shape[0], -s), xs.dtype), xs[:, :s]], axis=1)
        return xs

    def conv(xcm, w_ref_, s_, t_):
        zero = jnp.zeros_like(xcm)
        pick = {-1: jnp.where(right_edge, zero, xcm),
                0: xcm,
                1: jnp.where(left_edge, zero, xcm)}
        acc = jnp.zeros((cout, hw), jnp.float32)
        for dh_i, dh in enumerate((-1, 0, 1)):
            buf = xs_ref.at[dh_i % 2]
            for dw_i, dw in enumerate((-1, 0, 1)):
                buf[dw_i * c:(dw_i + 1) * c] = shift(pick[dw], dh * wo + dw)
            acc = acc + jnp.dot(w_ref_[dh_i], buf[...],
                                preferred_element_type=jnp.float32)
        return jnp.maximum(acc * s_ + t_, 0.0)

    for img in range(bsz):
        x_cm = x_ref[img].reshape(hw, c).T
        y1 = conv(x_cm, w1_ref, s1_ref[...], t1_ref[...])
        y2 = conv(y1.astype(jnp.bfloat16), w2_ref, s2_ref[...], t2_ref[...])
        o_ref[img] = y2


def kernel(x_nchw, w_up, b_up, conv1_w, conv1_scale, conv1_shift,
           conv2_w, conv2_scale, conv2_shift):
    n, cin, h, w = x_nchw.shape
    cout = w_up.shape[1]
    ho, wo = 2 * h, 2 * w

    w2d = jnp.transpose(w_up, (0, 2, 3, 1)).reshape(cin, 4 * cout)
    w2d = w2d.astype(jnp.bfloat16)
    b2d = jnp.tile(b_up, 4).reshape(1, 4 * cout)
    w1 = jnp.transpose(conv1_w, (0, 3, 1, 2)).reshape(3, cout, 3 * cout)
    w1 = w1.astype(jnp.bfloat16)
    w2 = jnp.transpose(conv2_w, (0, 3, 1, 2)).reshape(3, cout, 3 * cout)
    w2 = w2.astype(jnp.bfloat16)
    s1 = conv1_scale.reshape(cout, 1)
    t1 = conv1_shift.reshape(cout, 1)
    s2 = conv2_scale.reshape(cout, 1)
    t2 = conv2_shift.reshape(cout, 1)

    x_flat = x_nchw.reshape(n, cin, h * w)

    bsz = 2 if n % 2 == 0 else 1
    up = pl.pallas_call(
        _upsample_kernel,
        out_shape=jax.ShapeDtypeStruct((n, h, 2, w, 2 * cout), jnp.bfloat16),
        grid=(n // bsz,),
        in_specs=[
            pl.BlockSpec((bsz, cin, h * w), lambda i: (i, 0, 0)),
            pl.BlockSpec((cin, 4 * cout), lambda i: (0, 0)),
            pl.BlockSpec((1, 4 * cout), lambda i: (0, 0)),
        ],
        out_specs=pl.BlockSpec((bsz, h, 2, w, 2 * cout),
                               lambda i: (i, 0, 0, 0, 0)),
        compiler_params=pltpu.CompilerParams(
            dimension_semantics=("parallel",)),
    )(x_flat, w2d, b2d)

    out = pl.pallas_call(
        functools.partial(_double_conv_kernel, wo=wo),
        out_shape=jax.ShapeDtypeStruct((n, cout, ho * wo), jnp.float32),
        grid=(n // bsz,),
        in_specs=[
            pl.BlockSpec((bsz, h, 2, w, 2 * cout),
                         lambda i: (i, 0, 0, 0, 0)),
            pl.BlockSpec((3, cout, 3 * cout), lambda i: (0, 0, 0)),
            pl.BlockSpec((cout, 1), lambda i: (0, 0)),
            pl.BlockSpec((cout, 1), lambda i: (0, 0)),
            pl.BlockSpec((3, cout, 3 * cout), lambda i: (0, 0, 0)),
            pl.BlockSpec((cout, 1), lambda i: (0, 0)),
            pl.BlockSpec((cout, 1), lambda i: (0, 0)),
        ],
        out_specs=pl.BlockSpec((bsz, cout, ho * wo), lambda i: (i, 0, 0)),
        scratch_shapes=[pltpu.VMEM((2, 3 * cout, ho * wo), jnp.bfloat16)],
        compiler_params=pltpu.CompilerParams(
            dimension_semantics=("parallel",)),
    )(up, w1, s1, t1, w2, s2, t2)

    return out.reshape(n, cout, ho, wo)
